# Initial kernel scaffold; baseline (speedup 1.0000x reference)
import jax
import jax.numpy as jnp
from jax.experimental import pallas as pl


def kernel(walk_paths, node_embeddings, linear_w):
    raise NotImplementedError("write your pallas kernel here")


# speedup vs baseline: 1.6547x; 1.6547x over previous
import functools

import jax
import jax.numpy as jnp
from jax import lax
from jax.experimental import pallas as pl
from jax.experimental.pallas import tpu as pltpu
from jax.experimental.pallas import tpu_sc as plsc

HID = 64
WALK = 40
NC, NS, L = 2, 16, 16
NW = NC * NS

B, F = 1024, 26
M = B * F
ROWS_PER_W = M // NW
C = 2
CHUNKS = ROWS_PER_W // C
IDX_PER_W = ROWS_PER_W * WALK


def _make_kernel():
  mesh = plsc.VectorSubcoreMesh(core_axis_name="c", subcore_axis_name="s")

  @functools.partial(
      pl.kernel,
      mesh=mesh,
      compiler_params=pltpu.CompilerParams(use_tc_tiling_on_sc=False),
      out_type=jax.ShapeDtypeStruct((M, HID), jnp.float32),
      scratch_types=[
          pltpu.VMEM((IDX_PER_W,), jnp.int32),
          pltpu.VMEM((WALK * C, HID), jnp.float32),
          pltpu.VMEM((C, HID), jnp.float32),
          pltpu.SemaphoreType.DMA,
      ],
  )
  def body(idx_hbm, table_hbm, out_hbm, idx_v, rows_v, acc_v, sem):
    wid = lax.axis_index("s") * NC + lax.axis_index("c")
    row_base = wid * ROWS_PER_W
    pltpu.sync_copy(idx_hbm.at[pl.ds(row_base * WALK, IDX_PER_W)], idx_v)

    def chunk_body(ci, _):
      pltpu.async_copy(
          table_hbm.at[idx_v.at[pl.ds(ci * WALK * C, WALK * C)]],
          rows_v, sem).wait()
      for r in range(C):
        def red_body(j, carry):
          a0, a1, a2, a3 = carry
          rr = r * WALK + j
          a0 = a0 + rows_v[rr, pl.ds(0, L)]
          a1 = a1 + rows_v[rr, pl.ds(L, L)]
          a2 = a2 + rows_v[rr, pl.ds(2 * L, L)]
          a3 = a3 + rows_v[rr, pl.ds(3 * L, L)]
          return (a0, a1, a2, a3)

        z = jnp.zeros((L,), jnp.float32)
        a0, a1, a2, a3 = lax.fori_loop(0, WALK, red_body, (z, z, z, z))
        acc_v[r, pl.ds(0, L)] = a0
        acc_v[r, pl.ds(L, L)] = a1
        acc_v[r, pl.ds(2 * L, L)] = a2
        acc_v[r, pl.ds(3 * L, L)] = a3
      pltpu.sync_copy(acc_v, out_hbm.at[pl.ds(row_base + ci * C, C), :])
      return 0

    lax.fori_loop(0, CHUNKS, chunk_body, 0)

  return body


_sc_kernel = _make_kernel()


def kernel(walk_paths, node_embeddings, linear_w):
  del linear_w
  flat_idx = walk_paths.reshape(-1)
  out = _sc_kernel(flat_idx, node_embeddings)
  return out.reshape(B, F, HID)


# speedup vs baseline: 2.4035x; 1.4526x over previous
import functools

import jax
import jax.numpy as jnp
from jax import lax
from jax.experimental import pallas as pl
from jax.experimental.pallas import tpu as pltpu
from jax.experimental.pallas import tpu_sc as plsc

HID = 64
WALK = 40
NC, NS, L = 2, 16, 16
NW = NC * NS

B, F = 1024, 26
M = B * F
ROWS_PER_W = M // NW
C = 8
CHUNKS = ROWS_PER_W // C
IDX_PER_W = ROWS_PER_W * WALK
NBUF = 2


def _make_kernel():
  mesh = plsc.VectorSubcoreMesh(core_axis_name="c", subcore_axis_name="s")

  @functools.partial(
      pl.kernel,
      mesh=mesh,
      compiler_params=pltpu.CompilerParams(use_tc_tiling_on_sc=False),
      out_type=jax.ShapeDtypeStruct((M, HID), jnp.float32),
      scratch_types=[
          pltpu.VMEM((IDX_PER_W,), jnp.int32),
          [pltpu.VMEM((WALK * C, HID), jnp.float32) for _ in range(NBUF)],
          [pltpu.VMEM((C, HID), jnp.float32) for _ in range(NBUF)],
          [pltpu.SemaphoreType.DMA for _ in range(NBUF)],
          [pltpu.SemaphoreType.DMA for _ in range(NBUF)],
      ],
  )
  def body(idx_hbm, table_hbm, out_hbm, idx_v, rows_bufs, acc_bufs,
           gsems, osems):
    wid = lax.axis_index("s") * NC + lax.axis_index("c")
    row_base = wid * ROWS_PER_W
    pltpu.sync_copy(idx_hbm.at[pl.ds(row_base * WALK, IDX_PER_W)], idx_v)

    def start_gather(ci, b):
      pltpu.async_copy(
          table_hbm.at[idx_v.at[pl.ds(ci * WALK * C, WALK * C)]],
          rows_bufs[b], gsems[b])

    def wait_gather(b):
      pltpu.make_async_copy(
          table_hbm.at[idx_v.at[pl.ds(0, WALK * C)]],
          rows_bufs[b], gsems[b]).wait()

    def out_slice(ci):
      return out_hbm.at[pl.ds(row_base + ci * C, C), :]

    start_gather(0, 0)

    def outer(ci2, _):
      base_ci = ci2 * NBUF
      for b in range(NBUF):
        ci = base_ci + b
        nb = (b + 1) % NBUF

        @pl.when(ci + 1 < CHUNKS)
        def _():
          start_gather(ci + 1, nb)

        wait_gather(b)
        rows_v = rows_bufs[b]
        acc_v = acc_bufs[b]

        @pl.when(ci2 > 0)
        def _():
          pltpu.make_async_copy(acc_v, out_slice(ci), osems[b]).wait()

        for r in range(C):
          def red_body(jo, carry):
            a0, a1, a2, a3 = carry
            for ji in range(4):
              rr = r * WALK + jo * 4 + ji
              a0 = a0 + rows_v[rr, pl.ds(0, L)]
              a1 = a1 + rows_v[rr, pl.ds(L, L)]
              a2 = a2 + rows_v[rr, pl.ds(2 * L, L)]
              a3 = a3 + rows_v[rr, pl.ds(3 * L, L)]
            return (a0, a1, a2, a3)

          z = jnp.zeros((L,), jnp.float32)
          a0, a1, a2, a3 = lax.fori_loop(0, WALK // 4, red_body,
                                         (z, z, z, z))
          acc_v[r, pl.ds(0, L)] = a0
          acc_v[r, pl.ds(L, L)] = a1
          acc_v[r, pl.ds(2 * L, L)] = a2
          acc_v[r, pl.ds(3 * L, L)] = a3
        pltpu.async_copy(acc_v, out_slice(ci), osems[b])
      return 0

    lax.fori_loop(0, CHUNKS // NBUF, outer, 0)
    for b in range(NBUF):
      pltpu.make_async_copy(
          acc_bufs[b], out_slice(CHUNKS - NBUF + b), osems[b]).wait()

  return body


_sc_kernel = _make_kernel()


def kernel(walk_paths, node_embeddings, linear_w):
  del linear_w
  flat_idx = walk_paths.reshape(-1)
  out = _sc_kernel(flat_idx, node_embeddings)
  return out.reshape(B, F, HID)
